# baseline (speedup 1.0000x reference)
import jax
import jax.numpy as jnp
from jax import lax
from jax.experimental import pallas as pl
from jax.experimental.pallas import tpu as pltpu
from jax.experimental.pallas import tpu_sc as plsc

L_TOK = 2048
NBITS = 4096
NC = 2
NS = 16
NW = NC * NS
ROWS_PER_W = L_TOK // NW
R = 2
NGROUPS = ROWS_PER_W // R
LANES = 16
NCHUNK = NBITS // LANES


def _decoder_body(src_hbm, perm_hbm, out_hbm, perm_v,
                  rows0, rows1, rows2, outb0, outb1, outb2,
                  sin0, sin1, sin2, sout0, sout1, sout2, sperm):
    wid = lax.axis_index("s") * NC + lax.axis_index("c")
    rows = (rows0, rows1, rows2)
    outb = (outb0, outb1, outb2)
    sin = (sin0, sin1, sin2)
    sout = (sout0, sout1, sout2)
    out_base = wid * ROWS_PER_W
    src_base = L_TOK - out_base - ROWS_PER_W

    def in_copy(g, p):
        s0 = src_base + g * R
        return pltpu.make_async_copy(
            src_hbm.at[pl.ds(s0, R)], rows[p], sin[p])

    def out_copy(g, p):
        o0 = L_TOK - (src_base + g * R) - R
        return pltpu.make_async_copy(
            outb[p], out_hbm.at[pl.ds(o0, R)], sout[p])

    def gather_group(p):
        rbuf, obuf = rows[p], outb[p]

        @plsc.parallel_loop(0, NCHUNK, unroll=8)
        def _(k):
            col0 = k * LANES
            idx = perm_v[pl.ds(col0, LANES)]
            for r in range(R):
                row_sel = jnp.full((LANES,), r, jnp.int32)
                v = plsc.load_gather(rbuf, [row_sel, idx])
                obuf[R - 1 - r, pl.ds(col0, LANES)] = v

    perm_dma = pltpu.make_async_copy(perm_hbm, perm_v, sperm)
    perm_dma.start()
    in_copy(0, 0).start()
    in_copy(1, 1).start()
    in_copy(2, 2).start()
    perm_dma.wait()
    for g in range(NGROUPS):
        p = g % 3
        in_copy(g, p).wait()
        if g >= 3:
            out_copy(g - 3, p).wait()
        gather_group(p)
        out_copy(g, p).start()
        if g + 3 < NGROUPS:
            in_copy(g + 3, p).start()
    for g in range(NGROUPS - 3, NGROUPS):
        out_copy(g, g % 3).wait()


def kernel(source, perm):
    mesh = plsc.VectorSubcoreMesh(core_axis_name="c", subcore_axis_name="s")
    f = pl.kernel(
        _decoder_body,
        mesh=mesh,
        compiler_params=pltpu.CompilerParams(needs_layout_passes=False),
        out_type=jax.ShapeDtypeStruct((L_TOK, NBITS), jnp.float32),
        scratch_types=[
            pltpu.VMEM((NBITS,), jnp.int32),
            pltpu.VMEM((R, NBITS), jnp.float32),
            pltpu.VMEM((R, NBITS), jnp.float32),
            pltpu.VMEM((R, NBITS), jnp.float32),
            pltpu.VMEM((R, NBITS), jnp.float32),
            pltpu.VMEM((R, NBITS), jnp.float32),
            pltpu.VMEM((R, NBITS), jnp.float32),
            pltpu.SemaphoreType.DMA,
            pltpu.SemaphoreType.DMA,
            pltpu.SemaphoreType.DMA,
            pltpu.SemaphoreType.DMA,
            pltpu.SemaphoreType.DMA,
            pltpu.SemaphoreType.DMA,
            pltpu.SemaphoreType.DMA,
        ],
    )
    return f(source, perm)


# speedup vs baseline: 1.1096x; 1.1096x over previous
import jax
import jax.numpy as jnp
from jax import lax
from jax.experimental import pallas as pl
from jax.experimental.pallas import tpu as pltpu
from jax.experimental.pallas import tpu_sc as plsc

L_TOK = 2048
NBITS = 4096
NC = 2
NS = 16
NW = NC * NS
ROWS_PER_W = L_TOK // NW
R = 4
NGROUPS = ROWS_PER_W // R
LANES = 16
NCHUNK = NBITS // LANES


def _decoder_body(src_hbm, perm_hbm, out_hbm, perm_v,
                  rows0, rows1, rows2, rows3, outb0, outb1, outb2,
                  sin0, sin1, sin2, sin3, sout0, sout1, sout2, sperm):
    wid = lax.axis_index("s") * NC + lax.axis_index("c")
    rows = (rows0, rows1, rows2, rows3)
    outb = (outb0, outb1, outb2)
    sin = (sin0, sin1, sin2, sin3)
    sout = (sout0, sout1, sout2)
    out_base = wid * ROWS_PER_W
    src_base = L_TOK - out_base - ROWS_PER_W

    def in_copy(g, p):
        s0 = src_base + g * R
        return pltpu.make_async_copy(
            src_hbm.at[pl.ds(s0, R)], rows[p], sin[p])

    def out_copy(g, p):
        o0 = L_TOK - (src_base + g * R) - R
        return pltpu.make_async_copy(
            outb[p], out_hbm.at[pl.ds(o0, R)], sout[p])

    def gather_group(pi, po):
        rbuf, obuf = rows[pi], outb[po]

        @plsc.parallel_loop(0, NCHUNK, unroll=8)
        def _(k):
            col0 = k * LANES
            idx = perm_v[pl.ds(col0, LANES)]
            for r in range(R):
                row_sel = jnp.full((LANES,), r, jnp.int32)
                v = plsc.load_gather(rbuf, [row_sel, idx])
                obuf[R - 1 - r, pl.ds(col0, LANES)] = v

    perm_dma = pltpu.make_async_copy(perm_hbm, perm_v, sperm)
    perm_dma.start()
    in_copy(0, 0).start()
    in_copy(1, 1).start()
    in_copy(2, 2).start()
    in_copy(3, 3).start()
    perm_dma.wait()
    for g in range(NGROUPS):
        pi = g % 4
        po = g % 3
        in_copy(g, pi).wait()
        if g >= 3:
            out_copy(g - 3, po).wait()
        gather_group(pi, po)
        out_copy(g, po).start()
        if g + 4 < NGROUPS:
            in_copy(g + 4, pi).start()
    for g in range(NGROUPS - 3, NGROUPS):
        out_copy(g, g % 3).wait()


def kernel(source, perm):
    mesh = plsc.VectorSubcoreMesh(core_axis_name="c", subcore_axis_name="s")
    f = pl.kernel(
        _decoder_body,
        mesh=mesh,
        compiler_params=pltpu.CompilerParams(needs_layout_passes=False),
        out_type=jax.ShapeDtypeStruct((L_TOK, NBITS), jnp.float32),
        scratch_types=[
            pltpu.VMEM((NBITS,), jnp.int32),
            pltpu.VMEM((R, NBITS), jnp.float32),
            pltpu.VMEM((R, NBITS), jnp.float32),
            pltpu.VMEM((R, NBITS), jnp.float32),
            pltpu.VMEM((R, NBITS), jnp.float32),
            pltpu.VMEM((R, NBITS), jnp.float32),
            pltpu.VMEM((R, NBITS), jnp.float32),
            pltpu.VMEM((R, NBITS), jnp.float32),
            pltpu.SemaphoreType.DMA,
            pltpu.SemaphoreType.DMA,
            pltpu.SemaphoreType.DMA,
            pltpu.SemaphoreType.DMA,
            pltpu.SemaphoreType.DMA,
            pltpu.SemaphoreType.DMA,
            pltpu.SemaphoreType.DMA,
            pltpu.SemaphoreType.DMA,
        ],
    )
    return f(source, perm)
